# baseline (speedup 1.0000x reference)
import functools

import jax
import jax.numpy as jnp
from jax import lax
from jax.experimental import pallas as pl
from jax.experimental.pallas import tpu as pltpu
from jax.experimental.pallas import tpu_sc as plsc

N_CAT = 26
N_NUM = 13
VOCAB = 100000
DIM = 32
BATCH = 16384
EPS = 1e-5
EMB_F = N_CAT * DIM
FEAT = EMB_F + N_NUM

NROWS = BATCH * N_CAT
NW = 32
ROWS_W = NROWS // NW
IDXROW = 128
NCHUNK = 13
CHUNK = ROWS_W // NCHUNK
KPC = CHUNK // IDXROW


def _gather_sc(tables_flat, idx2d):
    mesh = plsc.VectorSubcoreMesh(core_axis_name="c", subcore_axis_name="s")

    @functools.partial(
        pl.kernel,
        mesh=mesh,
        out_type=jax.ShapeDtypeStruct((NROWS, DIM), jnp.float32),
        compiler_params=pltpu.CompilerParams(use_tc_tiling_on_sc=False),
        scratch_types=[
            pltpu.VMEM((KPC, IDXROW), jnp.int32),
            pltpu.VMEM((CHUNK, DIM), jnp.float32),
            pltpu.SemaphoreType.DMA,
        ],
    )
    def gather_kernel(table_hbm, idx_hbm, out_hbm, idx_v, rows_v, sem):
        wid = lax.axis_index("s") * 2 + lax.axis_index("c")

        def body(c, carry):
            row0 = wid * ROWS_W + c * CHUNK
            irow = wid * (ROWS_W // IDXROW) + c * KPC
            pltpu.sync_copy(idx_hbm.at[pl.ds(irow, KPC)], idx_v)
            copies = [
                pltpu.make_async_copy(
                    table_hbm.at[idx_v.at[j]],
                    rows_v.at[pl.ds(j * IDXROW, IDXROW)],
                    sem,
                )
                for j in range(KPC)
            ]
            for cp in copies:
                cp.start()
            for cp in copies:
                cp.wait()
            pltpu.sync_copy(rows_v, out_hbm.at[pl.ds(row0, CHUNK)])
            return carry

        lax.fori_loop(0, NCHUNK, body, 0)

    return gather_kernel(tables_flat, idx2d)


BS = 1024
NB = BATCH // BS


def _stats_tc(emb2, x_num, ge, gn, be, bn):
    def stats_kernel(emb_ref, num_ref, ge_ref, gn_ref, be_ref, bn_ref,
                     se_ref, sn_ref, he_ref, hn_ref, s1, s2, n1, n2):
        j = pl.program_id(0)
        e = emb_ref[...]
        x = num_ref[...]
        pe = jnp.sum(e, axis=0, keepdims=True)
        pe2 = jnp.sum(e * e, axis=0, keepdims=True)
        pn = jnp.sum(x, axis=0, keepdims=True)
        pn2 = jnp.sum(x * x, axis=0, keepdims=True)

        @pl.when(j == 0)
        def _():
            s1[...] = pe
            s2[...] = pe2
            n1[...] = pn
            n2[...] = pn2

        @pl.when(j > 0)
        def _():
            s1[...] += pe
            s2[...] += pe2
            n1[...] += pn
            n2[...] += pn2

        @pl.when(j == NB - 1)
        def _():
            inv_b = jnp.float32(1.0 / BATCH)
            me = s1[...] * inv_b
            ve = s2[...] * inv_b - me * me
            re = lax.rsqrt(ve + EPS)
            mn = n1[...] * inv_b
            vn = n2[...] * inv_b - mn * mn
            rn = lax.rsqrt(vn + EPS)
            sc_e = ge_ref[...] * re
            sc_n = gn_ref[...] * rn
            se_ref[...] = sc_e
            sn_ref[...] = sc_n
            he_ref[...] = be_ref[...] - me * sc_e
            hn_ref[...] = bn_ref[...] - mn * sc_n

    return pl.pallas_call(
        stats_kernel,
        grid=(NB,),
        in_specs=[
            pl.BlockSpec((BS, EMB_F), lambda j: (j, 0)),
            pl.BlockSpec((BS, N_NUM), lambda j: (j, 0)),
            pl.BlockSpec((1, EMB_F), lambda j: (0, 0)),
            pl.BlockSpec((1, N_NUM), lambda j: (0, 0)),
            pl.BlockSpec((1, EMB_F), lambda j: (0, 0)),
            pl.BlockSpec((1, N_NUM), lambda j: (0, 0)),
        ],
        out_specs=[
            pl.BlockSpec((1, EMB_F), lambda j: (0, 0)),
            pl.BlockSpec((1, N_NUM), lambda j: (0, 0)),
            pl.BlockSpec((1, EMB_F), lambda j: (0, 0)),
            pl.BlockSpec((1, N_NUM), lambda j: (0, 0)),
        ],
        out_shape=[
            jax.ShapeDtypeStruct((1, EMB_F), jnp.float32),
            jax.ShapeDtypeStruct((1, N_NUM), jnp.float32),
            jax.ShapeDtypeStruct((1, EMB_F), jnp.float32),
            jax.ShapeDtypeStruct((1, N_NUM), jnp.float32),
        ],
        scratch_shapes=[
            pltpu.VMEM((1, EMB_F), jnp.float32),
            pltpu.VMEM((1, EMB_F), jnp.float32),
            pltpu.VMEM((1, N_NUM), jnp.float32),
            pltpu.VMEM((1, N_NUM), jnp.float32),
        ],
    )(emb2, x_num, ge, gn, be, bn)


def _norm_tc(emb2, x_num, se, sn, he, hn):
    def norm_kernel(emb_ref, num_ref, se_ref, sn_ref, he_ref, hn_ref, out_ref):
        e = emb_ref[...] * se_ref[...] + he_ref[...]
        x = num_ref[...] * sn_ref[...] + hn_ref[...]
        out_ref[...] = jnp.concatenate([e, x], axis=1)

    return pl.pallas_call(
        norm_kernel,
        grid=(NB,),
        in_specs=[
            pl.BlockSpec((BS, EMB_F), lambda j: (j, 0)),
            pl.BlockSpec((BS, N_NUM), lambda j: (j, 0)),
            pl.BlockSpec((1, EMB_F), lambda j: (0, 0)),
            pl.BlockSpec((1, N_NUM), lambda j: (0, 0)),
            pl.BlockSpec((1, EMB_F), lambda j: (0, 0)),
            pl.BlockSpec((1, N_NUM), lambda j: (0, 0)),
        ],
        out_specs=pl.BlockSpec((BS, FEAT), lambda j: (j, 0)),
        out_shape=jax.ShapeDtypeStruct((BATCH, FEAT), jnp.float32),
    )(emb2, x_num, se, sn, he, hn)


def kernel(x_num, x_cat, tables, gamma, beta):
    x_cat = x_cat.astype(jnp.int32)
    idx2d = (x_cat + (jnp.arange(N_CAT, dtype=jnp.int32) * VOCAB)[None, :]
             ).reshape(NROWS // IDXROW, IDXROW)
    tflat = tables.reshape(N_CAT * VOCAB, DIM)
    emb = _gather_sc(tflat, idx2d)
    emb2 = emb.reshape(BATCH, EMB_F)
    ge = gamma[:EMB_F].reshape(1, EMB_F)
    gn = gamma[EMB_F:].reshape(1, N_NUM)
    be = beta[:EMB_F].reshape(1, EMB_F)
    bn = beta[EMB_F:].reshape(1, N_NUM)
    se, sn, he, hn = _stats_tc(emb2, x_num, ge, gn, be, bn)
    return _norm_tc(emb2, x_num, se, sn, he, hn)
